# baseline (speedup 1.0000x reference)
import functools

import jax
import jax.numpy as jnp
from jax import lax
from jax.experimental import pallas as pl
from jax.experimental.pallas import tpu as pltpu
from jax.experimental.pallas import tpu_sc as plsc

VOCAB = 100000
D_MODEL = 64
D_PAD = 65
BATCH = 4096
HIST = 200
B_TOTAL = BATCH * HIST

_INFO = plsc.get_sparse_core_info()
_NC = _INFO.num_cores
_NS = _INFO.num_subcores
_L = _INFO.num_lanes
_NW = _NC * _NS
_BT = BATCH // _NW
_B_PER_W = _BT * HIST
_DT = D_MODEL // 8


def _emb_body(tok_hbm, w_hbm, out_hbm, tok_v, tokT_v, gbuf, sbuf, tbuf,
              gsem, osem):
  wid = lax.axis_index("s") * _NC + lax.axis_index("c")
  pltpu.sync_copy(tok_hbm.at[pl.ds(wid * _B_PER_W, _B_PER_W)], tok_v)

  lane = lax.iota(jnp.int32, _L)
  rows = [lane + j * _L for j in range(_BT // _L)]
  zcol = lane * 0

  @plsc.parallel_loop(0, HIST, unroll=2)
  def _tok_t(h):
    vals = [plsc.load_gather(tok_v, [r * HIST + h]) for r in rows]
    for j, v in enumerate(vals):
      tokT_v[h, pl.ds(j * _L, _L)] = v

  def gather(h, s):
    return pltpu.make_async_copy(
        w_hbm.at[tokT_v.at[h]], gbuf.at[s], gsem.at[s])

  def store(h, s):
    return pltpu.make_async_copy(
        tbuf.at[s], out_hbm.at[h, :, wid], osem.at[s])

  gather(0, 0).start()
  gather(1, 1).start()

  @pl.loop(0, HIST)
  def _h(h):
    s = lax.rem(h, 2)
    gather(h, s).wait()

    @pl.when(h >= 2)
    def _drain():
      store(h - 2, s).wait()

    gsrc = gbuf.at[s]

    @plsc.parallel_loop(0, _BT, unroll=8)
    def _reskew(b):
      for j in range(D_MODEL // _L):
        sbuf[b, pl.ds(j * _L, _L)] = gsrc[b, pl.ds(j * _L, _L)]

    tdst = tbuf.at[s]

    @plsc.parallel_loop(0, D_MODEL, unroll=8)
    def _tp(d):
      dvec = zcol + d
      dt = lax.div(d, 8)
      di = lax.rem(d, 8)
      for j in range(_BT // _L):
        tdst[dt, di, pl.ds(j * _L, _L)] = plsc.load_gather(
            sbuf, [rows[j], dvec])

    store(h, s).start()

    @pl.when(h + 2 < HIST)
    def _next():
      gather(h + 2, s).start()

  store(HIST - 2, 0).wait()
  store(HIST - 1, 1).wait()


_emb = functools.partial(
    pl.kernel,
    out_type=jax.ShapeDtypeStruct((HIST, _DT, _NW, 8, 128), jnp.float32),
    mesh=plsc.VectorSubcoreMesh(core_axis_name="c", subcore_axis_name="s"),
    scratch_types=[
        pltpu.VMEM((_B_PER_W,), jnp.int32),
        pltpu.VMEM((HIST, _BT), jnp.int32),
        pltpu.VMEM((2, _BT, D_MODEL), jnp.float32),
        pltpu.VMEM((_BT, D_PAD), jnp.float32),
        pltpu.VMEM((2, _DT, 8, 128), jnp.float32),
        pltpu.SemaphoreType.DMA((2,)),
        pltpu.SemaphoreType.DMA((2,)),
    ],
    compiler_params=pltpu.CompilerParams(
        use_tc_tiling_on_sc=False, needs_layout_passes=False),
)(_emb_body)


@jax.jit
def kernel(token_ids, weight):
  tok = token_ids.reshape(B_TOTAL).astype(jnp.int32)
  out5 = _emb(tok, weight)
  return out5.transpose(2, 4, 0, 1, 3).reshape(BATCH, HIST, D_MODEL)


# speedup vs baseline: 1.0772x; 1.0772x over previous
import functools

import jax
import jax.numpy as jnp
from jax import lax
from jax.experimental import pallas as pl
from jax.experimental.pallas import tpu as pltpu
from jax.experimental.pallas import tpu_sc as plsc

VOCAB = 100000
D_MODEL = 64
D_PAD = 65
BATCH = 4096
HIST = 200
B_TOTAL = BATCH * HIST

_INFO = plsc.get_sparse_core_info()
_NC = _INFO.num_cores
_NS = _INFO.num_subcores
_L = _INFO.num_lanes
_NW = _NC * _NS
_BT = BATCH // _NW
_B_PER_W = _BT * HIST
_DT = D_MODEL // 8


def _emb_body(tok_hbm, w_hbm, out_hbm, tok_v, tokT_v, gbuf, sbuf, tbuf,
              gsem, osem):
  wid = lax.axis_index("s") * _NC + lax.axis_index("c")
  pltpu.sync_copy(tok_hbm.at[pl.ds(wid * _B_PER_W, _B_PER_W)], tok_v)

  lane = lax.iota(jnp.int32, _L)
  rows = [lane + j * _L for j in range(_BT // _L)]
  zcol = lane * 0

  @plsc.parallel_loop(0, HIST, unroll=2)
  def _tok_t(h):
    vals = [plsc.load_gather(tok_v, [r * HIST + h]) for r in rows]
    for j, v in enumerate(vals):
      tokT_v[h, pl.ds(j * _L, _L)] = v

  def gather(h, s):
    return pltpu.make_async_copy(
        w_hbm.at[tokT_v.at[h]], gbuf.at[s], gsem.at[s])

  def store(h, s):
    return pltpu.make_async_copy(
        tbuf.at[s], out_hbm.at[h, :, wid], osem.at[s])

  gather(0, 0).start()
  gather(1, 1).start()

  @pl.loop(0, HIST)
  def _h(h):
    s = lax.rem(h, 2)
    gather(h, s).wait()

    @pl.when(h >= 2)
    def _drain():
      store(h - 2, s).wait()

    gsrc = gbuf.at[s]

    @plsc.parallel_loop(0, _BT, unroll=4)
    def _reskew(b):
      for j in range(D_MODEL // _L):
        sbuf[b, pl.ds(j * _L, _L)] = gsrc[b, pl.ds(j * _L, _L)]

    tdst = tbuf.at[s]

    @plsc.parallel_loop(0, D_MODEL, unroll=4)
    def _tp(d):
      dvec = zcol + d
      dt = lax.div(d, 8)
      di = lax.rem(d, 8)
      for j in range(_BT // _L):
        tdst[dt, di, pl.ds(j * _L, _L)] = plsc.load_gather(
            sbuf, [rows[j], dvec])

    store(h, s).start()

    @pl.when(h + 2 < HIST)
    def _next():
      gather(h + 2, s).start()

  store(HIST - 2, 0).wait()
  store(HIST - 1, 1).wait()


_emb = functools.partial(
    pl.kernel,
    out_type=jax.ShapeDtypeStruct((HIST, _DT, _NW, 8, 128), jnp.float32),
    mesh=plsc.VectorSubcoreMesh(core_axis_name="c", subcore_axis_name="s"),
    scratch_types=[
        pltpu.VMEM((_B_PER_W,), jnp.int32),
        pltpu.VMEM((HIST, _BT), jnp.int32),
        pltpu.VMEM((2, _BT, D_MODEL), jnp.float32),
        pltpu.VMEM((_BT, D_PAD), jnp.float32),
        pltpu.VMEM((2, _DT, 8, 128), jnp.float32),
        pltpu.SemaphoreType.DMA((2,)),
        pltpu.SemaphoreType.DMA((2,)),
    ],
    compiler_params=pltpu.CompilerParams(
        use_tc_tiling_on_sc=False, needs_layout_passes=False),
)(_emb_body)


@jax.jit
def kernel(token_ids, weight):
  tok = token_ids.reshape(B_TOTAL).astype(jnp.int32)
  out5 = _emb(tok, weight)
  return out5.transpose(2, 4, 0, 1, 3).reshape(BATCH, HIST, D_MODEL)


# speedup vs baseline: 1.2668x; 1.1760x over previous
import functools

import jax
import jax.numpy as jnp
from jax import lax
from jax.experimental import pallas as pl
from jax.experimental.pallas import tpu as pltpu
from jax.experimental.pallas import tpu_sc as plsc

VOCAB = 100000
D_MODEL = 64
D_PAD = 65
BATCH = 4096
HIST = 200
B_TOTAL = BATCH * HIST

_INFO = plsc.get_sparse_core_info()
_NC = _INFO.num_cores
_NS = _INFO.num_subcores
_L = _INFO.num_lanes
_NW = _NC * _NS
_BT = BATCH // _NW
_B_PER_W = _BT * HIST
_DT = D_MODEL // 8


def _emb_body(tok_hbm, w_hbm, out_hbm, tok_v, tokT_v, gbuf, sbuf, tbuf,
              gsem, osem):
  wid = lax.axis_index("s") * _NC + lax.axis_index("c")
  pltpu.sync_copy(tok_hbm.at[pl.ds(wid * _B_PER_W, _B_PER_W)], tok_v)

  lane = lax.iota(jnp.int32, _L)
  rows = [lane + j * _L for j in range(_BT // _L)]
  zcol = lane * 0

  @plsc.parallel_loop(0, HIST, unroll=2)
  def _tok_t(h):
    vals = [plsc.load_gather(tok_v, [r * HIST + h]) for r in rows]
    for j, v in enumerate(vals):
      tokT_v[h, pl.ds(j * _L, _L)] = v

  def gather(h, s):
    return pltpu.make_async_copy(
        w_hbm.at[tokT_v.at[h]], gbuf.at[s], gsem.at[s])

  def store(h, s):
    return pltpu.make_async_copy(
        tbuf.at[s], out_hbm.at[h, :, wid], osem.at[s])

  gather(0, 0).start()
  gather(1, 1).start()
  gather(2, 2).start()

  @pl.loop(0, HIST)
  def _h(h):
    s = lax.rem(h, 3)
    gather(h, s).wait()

    @pl.when(h >= 3)
    def _drain():
      store(h - 3, s).wait()

    gsrc = gbuf.at[s]

    @plsc.parallel_loop(0, _BT, unroll=4)
    def _reskew(b):
      for j in range(D_MODEL // _L):
        sbuf[b, pl.ds(j * _L, _L)] = gsrc[b, pl.ds(j * _L, _L)]

    tdst = tbuf.at[s]

    @plsc.parallel_loop(0, D_MODEL, unroll=4)
    def _tp(d):
      dvec = zcol + d
      dt = lax.div(d, 8)
      di = lax.rem(d, 8)
      for j in range(_BT // _L):
        tdst[dt, di, pl.ds(j * _L, _L)] = plsc.load_gather(
            sbuf, [rows[j], dvec])

    store(h, s).start()

    @pl.when(h + 3 < HIST)
    def _next():
      gather(h + 3, s).start()

  for hh in range(HIST - 3, HIST):
    store(hh, hh % 3).wait()


_emb = functools.partial(
    pl.kernel,
    out_type=jax.ShapeDtypeStruct((HIST, _DT, _NW, 8, 128), jnp.float32),
    mesh=plsc.VectorSubcoreMesh(core_axis_name="c", subcore_axis_name="s"),
    scratch_types=[
        pltpu.VMEM((_B_PER_W,), jnp.int32),
        pltpu.VMEM((HIST, _BT), jnp.int32),
        pltpu.VMEM((3, _BT, D_MODEL), jnp.float32),
        pltpu.VMEM((_BT, D_PAD), jnp.float32),
        pltpu.VMEM((3, _DT, 8, 128), jnp.float32),
        pltpu.SemaphoreType.DMA((3,)),
        pltpu.SemaphoreType.DMA((3,)),
    ],
    compiler_params=pltpu.CompilerParams(
        use_tc_tiling_on_sc=False, needs_layout_passes=False),
)(_emb_body)


@jax.jit
def kernel(token_ids, weight):
  tok = token_ids.reshape(B_TOTAL).astype(jnp.int32)
  out5 = _emb(tok, weight)
  return out5.transpose(2, 4, 0, 1, 3).reshape(BATCH, HIST, D_MODEL)


# speedup vs baseline: 1.2898x; 1.0181x over previous
import functools

import jax
import jax.numpy as jnp
from jax import lax
from jax.experimental import pallas as pl
from jax.experimental.pallas import tpu as pltpu
from jax.experimental.pallas import tpu_sc as plsc

VOCAB = 100000
D_MODEL = 64
D_PAD = 65
BATCH = 4096
HIST = 200
B_TOTAL = BATCH * HIST

_INFO = plsc.get_sparse_core_info()
_NC = _INFO.num_cores
_NS = _INFO.num_subcores
_L = _INFO.num_lanes
_NW = _NC * _NS
_BT = BATCH // _NW
_B_PER_W = _BT * HIST
_DT = D_MODEL // 8


def _emb_body(tok_hbm, w_hbm, out_hbm, tok_v, tokT_v, gbuf, sbuf, tbuf,
              gsem, osem):
  wid = lax.axis_index("s") * _NC + lax.axis_index("c")
  pltpu.sync_copy(tok_hbm.at[pl.ds(wid * _B_PER_W, _B_PER_W)], tok_v)

  lane = lax.iota(jnp.int32, _L)
  rows = [lane + j * _L for j in range(_BT // _L)]
  zcol = lane * 0

  @plsc.parallel_loop(0, HIST, unroll=2)
  def _tok_t(h):
    vals = [plsc.load_gather(tok_v, [r * HIST + h]) for r in rows]
    for j, v in enumerate(vals):
      tokT_v[h, pl.ds(j * _L, _L)] = v

  def gather(h, s):
    return pltpu.make_async_copy(
        w_hbm.at[tokT_v.at[h]], gbuf.at[s], gsem.at[s])

  def store(h, s):
    return pltpu.make_async_copy(
        tbuf.at[s], out_hbm.at[h, :, wid], osem.at[s])

  for hp in range(4):
    gather(hp, hp).start()

  @pl.loop(0, HIST)
  def _h(h):
    s = lax.rem(h, 4)
    gather(h, s).wait()

    @pl.when(h >= 4)
    def _drain():
      store(h - 4, s).wait()

    gsrc = gbuf.at[s]

    @plsc.parallel_loop(0, _BT, unroll=4)
    def _reskew(b):
      for j in range(D_MODEL // _L):
        sbuf[b, pl.ds(j * _L, _L)] = gsrc[b, pl.ds(j * _L, _L)]

    tdst = tbuf.at[s]

    @plsc.parallel_loop(0, D_MODEL, unroll=4)
    def _tp(d):
      dvec = zcol + d
      dt = lax.div(d, 8)
      di = lax.rem(d, 8)
      for j in range(_BT // _L):
        tdst[dt, di, pl.ds(j * _L, _L)] = plsc.load_gather(
            sbuf, [rows[j], dvec])

    store(h, s).start()

    @pl.when(h + 4 < HIST)
    def _next():
      gather(h + 4, s).start()

  for hh in range(HIST - 4, HIST):
    store(hh, hh % 4).wait()


_emb = functools.partial(
    pl.kernel,
    out_type=jax.ShapeDtypeStruct((HIST, _DT, _NW, 8, 128), jnp.float32),
    mesh=plsc.VectorSubcoreMesh(core_axis_name="c", subcore_axis_name="s"),
    scratch_types=[
        pltpu.VMEM((_B_PER_W,), jnp.int32),
        pltpu.VMEM((HIST, _BT), jnp.int32),
        pltpu.VMEM((4, _BT, D_MODEL), jnp.float32),
        pltpu.VMEM((_BT, D_PAD), jnp.float32),
        pltpu.VMEM((4, _DT, 8, 128), jnp.float32),
        pltpu.SemaphoreType.DMA((4,)),
        pltpu.SemaphoreType.DMA((4,)),
    ],
    compiler_params=pltpu.CompilerParams(
        use_tc_tiling_on_sc=False, needs_layout_passes=False),
)(_emb_body)


@jax.jit
def kernel(token_ids, weight):
  tok = token_ids.reshape(B_TOTAL).astype(jnp.int32)
  out5 = _emb(tok, weight)
  return out5.transpose(2, 4, 0, 1, 3).reshape(BATCH, HIST, D_MODEL)
